# Initial kernel scaffold; baseline (speedup 1.0000x reference)
import jax
import jax.numpy as jnp
from jax.experimental import pallas as pl


def kernel(word_sequences, word_embedding):
    raise NotImplementedError("write your pallas kernel here")


# speedup vs baseline: 5.7837x; 5.7837x over previous
import functools

import jax
import jax.numpy as jnp
from jax import lax
from jax.experimental import pallas as pl
from jax.experimental.pallas import tpu as pltpu
from jax.experimental.pallas import tpu_sc as plsc

VOCAB = 100000
EMBED = 128
BATCH = 1024
SEQ = 200

_INFO = plsc.get_sparse_core_info()
NC = _INFO.num_cores
NS = _INFO.num_subcores
NW = NC * NS
N = BATCH * SEQ
PW = N // NW
CH = 128
K = PW // CH


def _body(table_hbm, idx_hbm, out_hbm, idx_v, rows_v, sem):
    wid = lax.axis_index("s") * NC + lax.axis_index("c")
    pltpu.sync_copy(idx_hbm.at[wid], idx_v)
    base = wid * PW

    def step(j, carry):
        pltpu.async_copy(table_hbm.at[idx_v.at[j]], rows_v, sem).wait()
        pltpu.sync_copy(rows_v, out_hbm.at[pl.ds(base + j * CH, CH)])
        return carry

    lax.fori_loop(0, K, step, 0)


@jax.jit
def _gather(table, idx):
    mesh = plsc.VectorSubcoreMesh(core_axis_name="c", subcore_axis_name="s")
    return pl.kernel(
        _body,
        out_type=jax.ShapeDtypeStruct((N, EMBED), jnp.float32),
        mesh=mesh,
        scratch_types=[
            pltpu.VMEM((K, CH), jnp.int32),
            pltpu.VMEM((CH, EMBED), jnp.float32),
            pltpu.SemaphoreType.DMA,
        ],
    )(table, idx)


def kernel(word_sequences, word_embedding):
    idx = word_sequences.astype(jnp.int32).reshape(NW, K, CH)
    out = _gather(word_embedding, idx)
    return out.reshape(BATCH, SEQ, EMBED)


# speedup vs baseline: 6.6076x; 1.1424x over previous
import functools

import jax
import jax.numpy as jnp
from jax import lax
from jax.experimental import pallas as pl
from jax.experimental.pallas import tpu as pltpu
from jax.experimental.pallas import tpu_sc as plsc

VOCAB = 100000
EMBED = 128
BATCH = 1024
SEQ = 200

_INFO = plsc.get_sparse_core_info()
NC = _INFO.num_cores
NS = _INFO.num_subcores
NW = NC * NS
N = BATCH * SEQ
PW = N // NW
CH = 128
K = PW // CH


def _body(table_hbm, idx_hbm, out_hbm, idx_v, rows0, rows1, gsem, ssem):
    wid = lax.axis_index("s") * NC + lax.axis_index("c")
    pltpu.sync_copy(idx_hbm.at[wid], idx_v)
    base = wid * PW
    bufs = (rows0, rows1)

    def gather(j, buf):
        pltpu.async_copy(table_hbm.at[idx_v.at[j]], buf, gsem)

    def scatter(j, buf):
        pltpu.async_copy(buf, out_hbm.at[pl.ds(base + j * CH, CH)], ssem)

    def wait(sem):
        pltpu.make_async_copy(rows0, out_hbm.at[pl.ds(base, CH)], sem).wait()

    gather(0, bufs[0])
    wait(gsem)
    gather(1, bufs[1])
    scatter(0, bufs[0])

    def pair(i, carry):
        j0 = 2 + 2 * i
        for b in range(2):
            j = j0 + b
            wait(gsem)
            wait(ssem)
            gather(j, bufs[b])
            scatter(j - 1, bufs[1 - b])
        return carry

    lax.fori_loop(0, (K - 2) // 2, pair, 0)

    wait(gsem)
    scatter(K - 1, bufs[1])
    wait(ssem)
    wait(ssem)


@jax.jit
def _gather(table, idx):
    mesh = plsc.VectorSubcoreMesh(core_axis_name="c", subcore_axis_name="s")
    return pl.kernel(
        _body,
        out_type=jax.ShapeDtypeStruct((N, EMBED), jnp.float32),
        mesh=mesh,
        scratch_types=[
            pltpu.VMEM((K, CH), jnp.int32),
            pltpu.VMEM((CH, EMBED), jnp.float32),
            pltpu.VMEM((CH, EMBED), jnp.float32),
            pltpu.SemaphoreType.DMA,
            pltpu.SemaphoreType.DMA,
        ],
    )(table, idx)


def kernel(word_sequences, word_embedding):
    idx = word_sequences.astype(jnp.int32).reshape(NW, K, CH)
    out = _gather(word_embedding, idx)
    return out.reshape(BATCH, SEQ, EMBED)


# speedup vs baseline: 6.6316x; 1.0036x over previous
import functools

import jax
import jax.numpy as jnp
from jax import lax
from jax.experimental import pallas as pl
from jax.experimental.pallas import tpu as pltpu
from jax.experimental.pallas import tpu_sc as plsc

VOCAB = 100000
EMBED = 128
BATCH = 1024
SEQ = 200

_INFO = plsc.get_sparse_core_info()
NC = _INFO.num_cores
NS = _INFO.num_subcores
NW = NC * NS
N = BATCH * SEQ
PW = N // NW
CH = 128
K = PW // CH


NBUF = 4


def _body(table_hbm, idx_hbm, out_hbm, idx_v, rows0, rows1, rows2, rows3,
          gsem, ssem):
    wid = lax.axis_index("s") * NC + lax.axis_index("c")
    pltpu.sync_copy(idx_hbm.at[wid], idx_v)
    base = wid * PW
    bufs = (rows0, rows1, rows2, rows3)

    def gather(j, buf):
        pltpu.async_copy(table_hbm.at[idx_v.at[j]], buf, gsem)

    def scatter(j, buf):
        pltpu.async_copy(buf, out_hbm.at[pl.ds(base + j * CH, CH)], ssem)

    def wait(sem):
        pltpu.make_async_copy(rows0, out_hbm.at[pl.ds(base, CH)], sem).wait()

    for j in range(NBUF):
        if j >= 1:
            wait(gsem)
            scatter(j - 1, bufs[j - 1])
        gather(j, bufs[j])
    for j in range(NBUF, NBUF + 2):
        wait(gsem)
        wait(ssem)
        gather(j, bufs[j % NBUF])
        scatter(j - 1, bufs[(j - 1) % NBUF])

    def group(i, carry):
        j0 = NBUF + 2 + NBUF * i
        for b in range(NBUF):
            j = j0 + b
            wait(gsem)
            wait(ssem)
            gather(j, bufs[(2 + b) % NBUF])
            scatter(j - 1, bufs[(1 + b) % NBUF])
        return carry

    lax.fori_loop(0, (K - NBUF - 2) // NBUF, group, 0)

    wait(gsem)
    scatter(K - 1, bufs[(K - 1) % NBUF])
    for _ in range(NBUF):
        wait(ssem)


@jax.jit
def _gather(table, idx):
    mesh = plsc.VectorSubcoreMesh(core_axis_name="c", subcore_axis_name="s")
    return pl.kernel(
        _body,
        out_type=jax.ShapeDtypeStruct((N, EMBED), jnp.float32),
        mesh=mesh,
        scratch_types=[
            pltpu.VMEM((K, CH), jnp.int32),
            pltpu.VMEM((CH, EMBED), jnp.float32),
            pltpu.VMEM((CH, EMBED), jnp.float32),
            pltpu.VMEM((CH, EMBED), jnp.float32),
            pltpu.VMEM((CH, EMBED), jnp.float32),
            pltpu.SemaphoreType.DMA,
            pltpu.SemaphoreType.DMA,
        ],
    )(table, idx)


def kernel(word_sequences, word_embedding):
    idx = word_sequences.astype(jnp.int32).reshape(NW, K, CH)
    out = _gather(word_embedding, idx)
    return out.reshape(BATCH, SEQ, EMBED)


# speedup vs baseline: 8.0510x; 1.2140x over previous
import functools

import jax
import jax.numpy as jnp
from jax import lax
from jax.experimental import pallas as pl
from jax.experimental.pallas import tpu as pltpu
from jax.experimental.pallas import tpu_sc as plsc

VOCAB = 100000
EMBED = 128
BATCH = 1024
SEQ = 200

_INFO = plsc.get_sparse_core_info()
NC = _INFO.num_cores
NS = _INFO.num_subcores
NW = NC * NS
N = BATCH * SEQ
PW = N // NW
CH = 128
K = PW // CH


NBUF = 4


def _body(table_hbm, idx_hbm, out_hbm, idx_v, rows0, rows1, rows2, rows3,
          gsem, ssem):
    wid = lax.axis_index("s") * NC + lax.axis_index("c")
    pltpu.sync_copy(idx_hbm.at[wid], idx_v)
    base = wid * PW
    bufs = (rows0, rows1, rows2, rows3)

    def gather(j, buf):
        pltpu.async_copy(table_hbm.at[idx_v.at[j]], buf, gsem)

    def scatter(j, buf):
        pltpu.async_copy(buf, out_hbm.at[pl.ds(base + j * CH, CH)], ssem)

    def wait(sem):
        pltpu.make_async_copy(rows0, out_hbm.at[pl.ds(base, CH)], sem).wait()

    gather(0, bufs[0])
    gather(1, bufs[1])
    for j in range(4):
        wait(gsem)
        if j + 2 < NBUF:
            gather(j + 2, bufs[j + 2])
        else:
            wait(ssem)
            gather(j + 2, bufs[(j + 2) % NBUF])
        scatter(j, bufs[j % NBUF])

    def group(i, carry):
        j0 = 4 + NBUF * i
        for b in range(NBUF):
            j = j0 + b
            wait(gsem)
            wait(ssem)
            gather(j + 2, bufs[(2 + b) % NBUF])
            scatter(j, bufs[b])
        return carry

    lax.fori_loop(0, (K - 6) // NBUF, group, 0)

    for j in range(K - 2, K):
        wait(gsem)
        wait(ssem)
        scatter(j, bufs[j % NBUF])
    wait(ssem)
    wait(ssem)


@jax.jit
def _gather(table, idx):
    mesh = plsc.VectorSubcoreMesh(core_axis_name="c", subcore_axis_name="s")
    return pl.kernel(
        _body,
        out_type=jax.ShapeDtypeStruct((N, EMBED), jnp.float32),
        mesh=mesh,
        scratch_types=[
            pltpu.VMEM((K, CH), jnp.int32),
            pltpu.VMEM((CH, EMBED), jnp.float32),
            pltpu.VMEM((CH, EMBED), jnp.float32),
            pltpu.VMEM((CH, EMBED), jnp.float32),
            pltpu.VMEM((CH, EMBED), jnp.float32),
            pltpu.SemaphoreType.DMA,
            pltpu.SemaphoreType.DMA,
        ],
    )(table, idx)


def kernel(word_sequences, word_embedding):
    idx = word_sequences.astype(jnp.int32).reshape(NW, K, CH)
    out = _gather(word_embedding, idx)
    return out.reshape(BATCH, SEQ, EMBED)
